# baseline (speedup 1.0000x reference)
import jax
import jax.numpy as jnp
from jax.experimental import pallas as pl

NUM_CLASSES = 1000
BATCH = 16384
ROWS_PER_BLOCK = 1024
NUM_BLOCKS = BATCH // ROWS_PER_BLOCK


def _onehot_block(x_ref, o_ref):
    xv = x_ref[0, 0, :].reshape(ROWS_PER_BLOCK, 1)
    cols = jax.lax.broadcasted_iota(jnp.int32, (ROWS_PER_BLOCK, NUM_CLASSES), 1)
    o_ref[...] = (xv == cols).astype(jnp.float32)


def kernel(x, W, b):
    x3 = x.astype(jnp.int32).reshape(NUM_BLOCKS, 1, ROWS_PER_BLOCK)
    out = pl.pallas_call(
        _onehot_block,
        grid=(NUM_BLOCKS,),
        in_specs=[pl.BlockSpec((1, 1, ROWS_PER_BLOCK), lambda i: (i, 0, 0))],
        out_specs=pl.BlockSpec((ROWS_PER_BLOCK, NUM_CLASSES), lambda i: (i, 0)),
        out_shape=jax.ShapeDtypeStruct((BATCH, NUM_CLASSES), jnp.float32),
    )(x3)
    return out
